# baseline (speedup 1.0000x reference)
import functools

import jax
import jax.numpy as jnp
from jax.experimental import pallas as pl
from jax.experimental.pallas import tpu as pltpu

CODEBOOK_SIZE = 512
DIM = 32
BETA = 0.25
_GROUPS = CODEBOOK_SIZE // 8


def _vq_kernel(x_ref, cb_ref, quant_ref, idx_ref, loss_ref, m_ref, c2_ref):
    b = pl.program_id(0)
    xb = x_ref[0]
    cb = cb_ref[...]
    tlen = xb.shape[1]
    m_ref[...] = jax.lax.dot_general(
        cb * jnp.float32(-2.0), xb, (((1,), (0,)), ((), ())),
        precision=jax.lax.Precision.DEFAULT,
        preferred_element_type=jnp.float32,
    )
    a = jnp.sum(xb * xb, axis=0, keepdims=True)
    c2_ref[...] = jnp.sum(cb * cb, axis=1)[:, None]

    siota = jax.lax.broadcasted_iota(jnp.int32, (8, tlen), 0)
    big = jnp.float32(jnp.inf)

    def body(g, carry):
        run_min, run_g = carry
        mg = m_ref[pl.ds(g * 8, 8), :]
        d = (a + mg) + c2_ref[pl.ds(g * 8, 8), :]
        better = d < run_min
        run_min = jnp.minimum(run_min, d)
        run_g = jnp.where(better, g, run_g)
        return run_min, run_g

    run_min, run_g = jax.lax.fori_loop(
        0, _GROUPS, body,
        (jnp.full((8, tlen), big, jnp.float32),
         jnp.zeros((8, tlen), jnp.int32)),
    )
    run_j = run_g * 8 + siota
    gmin = jnp.min(run_min, axis=0, keepdims=True)
    idx = jnp.min(jnp.where(run_min == gmin, run_j, CODEBOOK_SIZE),
                  axis=0)
    idx_ref[0, 0, :] = idx

    kiota = jax.lax.broadcasted_iota(jnp.int32, (CODEBOOK_SIZE, tlen), 0)
    onehot = (kiota == idx[None, :]).astype(jnp.float32)
    quant = jax.lax.dot_general(
        cb, onehot, (((0,), (0,)), ((), ())),
        precision=jax.lax.Precision.HIGHEST,
        preferred_element_type=jnp.float32,
    )
    quant_ref[0] = quant
    diff = quant - xb
    part = jnp.sum(diff * diff).reshape(1, 1)
    @pl.when(b == 0)
    def _():
        loss_ref[...] = jnp.zeros_like(loss_ref)
    loss_ref[...] += part


@functools.partial(jax.jit, static_argnames=())
def kernel(x, codebook):
    bsz, dim, tlen = x.shape
    quant, idx3, loss_sum = pl.pallas_call(
        _vq_kernel,
        grid=(bsz,),
        in_specs=[
            pl.BlockSpec((1, dim, tlen), lambda b: (b, 0, 0)),
            pl.BlockSpec((CODEBOOK_SIZE, dim), lambda b: (0, 0)),
        ],
        out_specs=[
            pl.BlockSpec((1, dim, tlen), lambda b: (b, 0, 0)),
            pl.BlockSpec((1, 1, tlen), lambda b: (b, 0, 0)),
            pl.BlockSpec((1, 1), lambda b: (0, 0)),
        ],
        out_shape=[
            jax.ShapeDtypeStruct((bsz, dim, tlen), jnp.float32),
            jax.ShapeDtypeStruct((bsz, 1, tlen), jnp.int32),
            jax.ShapeDtypeStruct((1, 1), jnp.float32),
        ],
        scratch_shapes=[pltpu.VMEM((CODEBOOK_SIZE, tlen), jnp.float32),
                        pltpu.VMEM((CODEBOOK_SIZE, 1), jnp.float32)],
    )(x, codebook)
    idx = idx3.reshape(bsz, tlen)
    loss_vq = loss_sum[0, 0] / jnp.float32(bsz * dim * tlen)
    loss_commit = jnp.float32(BETA) * loss_vq
    return (quant, idx, loss_vq, loss_commit)


# speedup vs baseline: 5.4340x; 5.4340x over previous
import functools

import jax
import jax.numpy as jnp
from jax.experimental import pallas as pl
from jax.experimental.pallas import tpu as pltpu

CODEBOOK_SIZE = 512
DIM = 32
BETA = 0.25
_G = 8
_GROUPS = CODEBOOK_SIZE // _G


def _vq_kernel(x_ref, cb_ref, cbt_ref, quant_ref, idx_ref, loss_ref, m_ref):
    b = pl.program_id(0)
    xb = x_ref[0]
    cb = cb_ref[...]
    tlen = xb.shape[1]
    m_ref[...] = jax.lax.dot_general(
        cb * jnp.float32(-2.0), xb, (((1,), (0,)), ((), ())),
        precision=jax.lax.Precision.DEFAULT,
        preferred_element_type=jnp.float32,
    )
    a = jnp.sum(xb * xb, axis=0, keepdims=True)
    c2 = jnp.sum(cb * cb, axis=1)[:, None]

    run_min = jnp.full((_G, tlen), jnp.inf, jnp.float32)
    run_g = jnp.zeros((_G, tlen), jnp.int32)
    for g in range(_GROUPS):
        mg = m_ref[g * _G:(g + 1) * _G, :]
        d = (a + mg) + c2[g * _G:(g + 1) * _G]
        better = d < run_min
        run_min = jnp.minimum(run_min, d)
        run_g = jnp.where(better, g, run_g)
    siota = jax.lax.broadcasted_iota(jnp.int32, (_G, tlen), 0)
    run_j = run_g * _G + siota
    gmin = jnp.min(run_min, axis=0, keepdims=True)
    idx = jnp.min(jnp.where(run_min == gmin, run_j, CODEBOOK_SIZE),
                  axis=0)
    idx_ref[0, 0, :] = idx

    idx_lo = jnp.broadcast_to((idx & 127)[None, :], (DIM, tlen))
    chunk = jnp.broadcast_to((idx >> 7)[None, :], (DIM, tlen))
    quant = jnp.zeros((DIM, tlen), jnp.float32)
    for c in range(CODEBOOK_SIZE // 128):
        part = jnp.take_along_axis(
            cbt_ref[:, c * 128:(c + 1) * 128], idx_lo,
            axis=1, mode="promise_in_bounds",
        )
        quant = jnp.where(chunk == c, part, quant)
    quant_ref[0] = quant
    diff = quant - xb
    part = jnp.sum(diff * diff).reshape(1, 1)
    @pl.when(b == 0)
    def _():
        loss_ref[...] = jnp.zeros_like(loss_ref)
    loss_ref[...] += part


@functools.partial(jax.jit, static_argnames=())
def kernel(x, codebook):
    bsz, dim, tlen = x.shape
    quant, idx3, loss_sum = pl.pallas_call(
        _vq_kernel,
        grid=(bsz,),
        in_specs=[
            pl.BlockSpec((1, dim, tlen), lambda b: (b, 0, 0)),
            pl.BlockSpec((CODEBOOK_SIZE, dim), lambda b: (0, 0)),
            pl.BlockSpec((dim, CODEBOOK_SIZE), lambda b: (0, 0)),
        ],
        out_specs=[
            pl.BlockSpec((1, dim, tlen), lambda b: (b, 0, 0)),
            pl.BlockSpec((1, 1, tlen), lambda b: (b, 0, 0)),
            pl.BlockSpec((1, 1), lambda b: (0, 0)),
        ],
        out_shape=[
            jax.ShapeDtypeStruct((bsz, dim, tlen), jnp.float32),
            jax.ShapeDtypeStruct((bsz, 1, tlen), jnp.int32),
            jax.ShapeDtypeStruct((1, 1), jnp.float32),
        ],
        scratch_shapes=[pltpu.VMEM((CODEBOOK_SIZE, tlen), jnp.float32)],
    )(x, codebook, codebook.T)
    idx = idx3.reshape(bsz, tlen)
    loss_vq = loss_sum[0, 0] / jnp.float32(bsz * dim * tlen)
    loss_commit = jnp.float32(BETA) * loss_vq
    return (quant, idx, loss_vq, loss_commit)
